# Initial kernel scaffold; baseline (speedup 1.0000x reference)
import jax
import jax.numpy as jnp
from jax.experimental import pallas as pl


def kernel(hidden_states, gate_w, w_in, w_out, bias):
    raise NotImplementedError("write your pallas kernel here")


# speedup vs baseline: 3.0784x; 3.0784x over previous
import jax
import jax.numpy as jnp
from jax.experimental import pallas as pl
from jax.experimental.pallas import tpu as pltpu

_HIDDEN = 1024
_FFN = 1024
_E = 64
_TOP_K = 2
_T = 2048
_BM = 128
_NT = 96
_P = _NT * _BM
_PAIRS = _T * _TOP_K


def _router_body(x_ref, gw_ref, idx_ref, gate_ref):
    x = x_ref[...]
    gw = gw_ref[...]
    logits = jax.lax.dot_general(
        x, gw, (((1,), (1,)), ((), ())), preferred_element_type=jnp.float32
    )
    ids = jax.lax.broadcasted_iota(jnp.int32, logits.shape, 1)
    m1 = jnp.max(logits, axis=1, keepdims=True)
    i1 = jnp.min(jnp.where(logits == m1, ids, _E), axis=1, keepdims=True)
    masked = jnp.where(ids == i1, -jnp.inf, logits)
    m2 = jnp.max(masked, axis=1, keepdims=True)
    i2 = jnp.min(jnp.where(masked == m2, ids, _E), axis=1, keepdims=True)
    z = jnp.exp(m2 - m1)
    den = 1.0 + z
    idx_ref[:, 0:1] = i1
    idx_ref[:, 1:2] = i2
    gate_ref[:, 0:1] = 1.0 / den
    gate_ref[:, 1:2] = z / den


def _mlp_body(te_ref, nu_ref, x_ref, wi_ref, wo_ref, y_ref):
    del te_ref
    i = pl.program_id(0)

    @pl.when(i < nu_ref[0])
    def _():
        x = x_ref[...]
        h = jnp.dot(x, wi_ref[0], preferred_element_type=jnp.float32)
        h = h * jax.nn.sigmoid(h)
        y_ref[...] = jnp.dot(h, wo_ref[0], preferred_element_type=jnp.float32)


def kernel(hidden_states, gate_w, w_in, w_out, bias):
    top_idx, gates = pl.pallas_call(
        _router_body,
        out_shape=(
            jax.ShapeDtypeStruct((_T, _TOP_K), jnp.int32),
            jax.ShapeDtypeStruct((_T, _TOP_K), jnp.float32),
        ),
    )(hidden_states, gate_w)

    e_flat = top_idx.reshape(-1)
    order = jnp.argsort(e_flat).astype(jnp.int32)
    e_sorted = e_flat[order]
    counts = jnp.bincount(e_flat, length=_E).astype(jnp.int32)
    csum = jnp.cumsum(counts) - counts
    pcounts = ((counts + _BM - 1) // _BM) * _BM
    pend = jnp.cumsum(pcounts)
    pstart = pend - pcounts
    n_used = (pend[-1] // _BM).astype(jnp.int32).reshape((1,))
    pos_sorted = pstart[e_sorted] + (jnp.arange(_PAIRS, dtype=jnp.int32) - csum[e_sorted])
    src_pair = jnp.zeros((_P,), jnp.int32).at[pos_sorted].set(order)
    pair_pos = jnp.zeros((_PAIRS,), jnp.int32).at[order].set(pos_sorted)
    gp = pair_pos.reshape(_T, _TOP_K)
    tile_expert = jnp.searchsorted(
        pend, jnp.arange(_NT, dtype=jnp.int32) * _BM, side="right"
    ).astype(jnp.int32)
    tile_expert = jnp.minimum(tile_expert, _E - 1)

    x_padded = hidden_states[src_pair // _TOP_K]

    grid_spec = pltpu.PrefetchScalarGridSpec(
        num_scalar_prefetch=2,
        grid=(_NT,),
        in_specs=[
            pl.BlockSpec(
                (_BM, _HIDDEN), lambda i, te, nu: (jnp.minimum(i, nu[0] - 1), 0)
            ),
            pl.BlockSpec(
                (1, _HIDDEN, _FFN),
                lambda i, te, nu: (te[jnp.minimum(i, nu[0] - 1)], 0, 0),
            ),
            pl.BlockSpec(
                (1, _FFN, _HIDDEN),
                lambda i, te, nu: (te[jnp.minimum(i, nu[0] - 1)], 0, 0),
            ),
        ],
        out_specs=pl.BlockSpec(
            (_BM, _HIDDEN), lambda i, te, nu: (jnp.minimum(i, nu[0] - 1), 0)
        ),
    )
    y_padded = pl.pallas_call(
        _mlp_body,
        grid_spec=grid_spec,
        out_shape=jax.ShapeDtypeStruct((_P, _HIDDEN), jnp.float32),
        compiler_params=pltpu.CompilerParams(
            dimension_semantics=("arbitrary",)
        ),
    )(tile_expert, n_used, x_padded, w_in, w_out)

    out = (
        y_padded[gp[:, 0]] * gates[:, 0:1]
        + y_padded[gp[:, 1]] * gates[:, 1:2]
        + bias
    )
    return out


# speedup vs baseline: 4.4571x; 1.4479x over previous
import jax
import jax.numpy as jnp
from jax.experimental import pallas as pl
from jax.experimental.pallas import tpu as pltpu

_HIDDEN = 1024
_FFN = 1024
_E = 64
_TOP_K = 2
_T = 2048
_BM = 128
_NT = 96
_P = _NT * _BM
_PAIRS = _T * _TOP_K
_NB = _PAIRS // _BM


def _router_body(x_ref, gw_ref, pos_ref, gate_ref, te_ref, nu_ref):
    x = x_ref[...]
    gw = gw_ref[...]
    logits = jax.lax.dot_general(
        x, gw, (((1,), (1,)), ((), ())), preferred_element_type=jnp.float32
    )
    ids = jax.lax.broadcasted_iota(jnp.int32, logits.shape, 1)
    m1 = jnp.max(logits, axis=1, keepdims=True)
    i1 = jnp.min(jnp.where(logits == m1, ids, _E), axis=1, keepdims=True)
    masked = jnp.where(ids == i1, -jnp.inf, logits)
    m2 = jnp.max(masked, axis=1, keepdims=True)
    i2 = jnp.min(jnp.where(masked == m2, ids, _E), axis=1, keepdims=True)
    z = jnp.exp(m2 - m1)
    den = 1.0 + z
    gate_ref[:, 0:1] = 1.0 / den
    gate_ref[:, 1:2] = z / den

    e_pairs = jnp.concatenate([i1, i2], axis=0)
    lane_e = jax.lax.broadcasted_iota(jnp.int32, (_BM, _E), 1)
    r128 = jax.lax.broadcasted_iota(jnp.int32, (_BM, _BM), 0)
    c128 = jax.lax.broadcasted_iota(jnp.int32, (_BM, _BM), 1)
    ltri = jnp.where(c128 <= r128, 1.0, 0.0)
    ohs = []
    cums = []
    run = jnp.zeros((1, _E), jnp.float32)
    for b in range(_NB):
        e_b = e_pairs[b * _BM : (b + 1) * _BM]
        oh_b = (jnp.broadcast_to(e_b, (_BM, _E)) == lane_e).astype(jnp.float32)
        cum_b = (
            jnp.dot(ltri, oh_b, preferred_element_type=jnp.float32)
            + run
        )
        run = cum_b[_BM - 1 :, :]
        ohs.append(oh_b)
        cums.append(cum_b)
    counts = run
    pcounts = (((counts.astype(jnp.int32) + (_BM - 1)) // _BM) * _BM).astype(
        jnp.float32
    )
    rE = jax.lax.broadcasted_iota(jnp.int32, (_E, _E), 0)
    cE = jax.lax.broadcasted_iota(jnp.int32, (_E, _E), 1)
    utri = jnp.where(rE <= cE, 1.0, 0.0)
    pend = jnp.dot(pcounts, utri, preferred_element_type=jnp.float32)
    pstart = pend - pcounts
    nu_ref[...] = (pend[:, _E - 1 :].astype(jnp.int32)) // _BM
    for b in range(_NB):
        pos_b = jnp.sum(
            ohs[b] * (jnp.broadcast_to(pstart, (_BM, _E)) + cums[b]),
            axis=1,
            keepdims=True,
        ) - 1.0
        pos_ref[b * _BM : (b + 1) * _BM, :] = pos_b.astype(jnp.int32)

    tstart = jax.lax.broadcasted_iota(jnp.int32, (_NT, _E), 0) * _BM
    te = jnp.sum(
        jnp.where(
            jnp.broadcast_to(pend, (_NT, _E)).astype(jnp.int32) <= tstart, 1, 0
        ),
        axis=1,
        keepdims=True,
    )
    te_ref[...] = jnp.minimum(te, _E - 1).astype(jnp.int32)


def _mlp_body(te_ref, nu_ref, x_ref, wi_ref, wo_ref, y_ref):
    del te_ref
    i = pl.program_id(0)

    @pl.when(i < nu_ref[0])
    def _():
        x = x_ref[...]
        h = jnp.dot(x, wi_ref[0], preferred_element_type=jnp.float32)
        h = h * jax.nn.sigmoid(h)
        y_ref[...] = jnp.dot(h, wo_ref[0], preferred_element_type=jnp.float32)


def kernel(hidden_states, gate_w, w_in, w_out, bias):
    pos, gates, tile_expert, n_used = pl.pallas_call(
        _router_body,
        out_shape=(
            jax.ShapeDtypeStruct((_PAIRS, 1), jnp.int32),
            jax.ShapeDtypeStruct((_T, _TOP_K), jnp.float32),
            jax.ShapeDtypeStruct((_NT, 1), jnp.int32),
            jax.ShapeDtypeStruct((1, 1), jnp.int32),
        ),
    )(hidden_states, gate_w)
    pos2 = pos.reshape(_TOP_K, _T)
    tile_expert = tile_expert.reshape(_NT)
    n_used = n_used.reshape(1)

    x_padded = (
        jnp.zeros((_P, _HIDDEN), jnp.float32)
        .at[pos2[0]]
        .set(hidden_states, unique_indices=True, mode="promise_in_bounds")
        .at[pos2[1]]
        .set(hidden_states, unique_indices=True, mode="promise_in_bounds")
    )

    grid_spec = pltpu.PrefetchScalarGridSpec(
        num_scalar_prefetch=2,
        grid=(_NT,),
        in_specs=[
            pl.BlockSpec(
                (_BM, _HIDDEN), lambda i, te, nu: (jnp.minimum(i, nu[0] - 1), 0)
            ),
            pl.BlockSpec(
                (1, _HIDDEN, _FFN),
                lambda i, te, nu: (te[jnp.minimum(i, nu[0] - 1)], 0, 0),
            ),
            pl.BlockSpec(
                (1, _FFN, _HIDDEN),
                lambda i, te, nu: (te[jnp.minimum(i, nu[0] - 1)], 0, 0),
            ),
        ],
        out_specs=pl.BlockSpec(
            (_BM, _HIDDEN), lambda i, te, nu: (jnp.minimum(i, nu[0] - 1), 0)
        ),
    )
    y_padded = pl.pallas_call(
        _mlp_body,
        grid_spec=grid_spec,
        out_shape=jax.ShapeDtypeStruct((_P, _HIDDEN), jnp.float32),
        compiler_params=pltpu.CompilerParams(
            dimension_semantics=("arbitrary",)
        ),
    )(tile_expert, n_used, x_padded, w_in, w_out)

    out = (
        y_padded[pos2[0]] * gates[:, 0:1]
        + y_padded[pos2[1]] * gates[:, 1:2]
        + bias
    )
    return out
